# baseline (speedup 1.0000x reference)
import functools

import jax
import jax.numpy as jnp
from jax import lax
from jax.experimental import pallas as pl
from jax.experimental.pallas import tpu as pltpu
from jax.experimental.pallas import tpu_sc as plsc

V = 1_000_000
F = 128
B = 4096
EPS = 1e-5

_NC, _NS = 2, 16
_NW = _NC * _NS
_PW = (B * F) // _NW
_CH = 64
_NCHUNK = _PW // _CH
_NB = 8
_IR = 128
_L = 16
_G = _CH // _L


@functools.cache
def _make_sc_pool():
    mesh = plsc.VectorSubcoreMesh(core_axis_name="c", subcore_axis_name="s")

    @functools.partial(
        pl.kernel,
        out_type=jax.ShapeDtypeStruct((B, F), jnp.float32),
        mesh=mesh,
        compiler_params=pltpu.CompilerParams(needs_layout_passes=False),
        scratch_types=[
            pltpu.VMEM((_PW // _IR, _IR), jnp.int32),
            pltpu.VMEM((_CH, F), jnp.float32),
            pltpu.VMEM((_CH, F), jnp.float32),
            pltpu.VMEM((_CH, F), jnp.float32),
            pltpu.VMEM((_CH, F), jnp.float32),
            pltpu.VMEM((_CH, F), jnp.float32),
            pltpu.VMEM((_CH, F), jnp.float32),
            pltpu.VMEM((_CH, F), jnp.float32),
            pltpu.VMEM((_CH, F), jnp.float32),
            pltpu.VMEM((_PW // _IR, _IR), jnp.float32),
            pltpu.SemaphoreType.DMA,
            pltpu.SemaphoreType.DMA,
            pltpu.SemaphoreType.DMA,
            pltpu.SemaphoreType.DMA,
            pltpu.SemaphoreType.DMA,
            pltpu.SemaphoreType.DMA,
            pltpu.SemaphoreType.DMA,
            pltpu.SemaphoreType.DMA,
        ],
    )
    def _sc_pool(table_hbm, idx_hbm, out_hbm, idx_v,
                 r0, r1, r2, r3, r4, r5, r6, r7, out_v,
                 s0, s1, s2, s3, s4, s5, s6, s7):
        sems = (s0, s1, s2, s3, s4, s5, s6, s7)
        rows = (r0, r1, r2, r3, r4, r5, r6, r7)
        wid = lax.axis_index("s") * _NC + lax.axis_index("c")
        pltpu.sync_copy(idx_hbm.at[pl.ds(wid * (_PW // _IR), _PW // _IR)],
                        idx_v)
        cpr = _IR // _CH

        def idx_slice(c):
            return idx_v.at[c // cpr, pl.ds((c % cpr) * _CH, _CH)]

        lanes = lax.broadcasted_iota(jnp.int32, (_L,), 0)

        def take16(v, idx):
            return jnp.take_along_axis(v, idx, axis=0,
                                       mode="promise_in_bounds")

        def merge(a, b, s):
            sel = (lanes & s) == 0
            return jnp.where(sel, a + take16(a, lanes ^ s),
                             b + take16(b, lanes ^ s))

        for b in range(_NB):
            pltpu.async_copy(
                table_hbm.at[idx_slice(b)], rows[b], sems[b]
            )

        def outer(co, carry):
            for b in range(_NB):
                c = co * _NB + b
                pltpu.make_async_copy(
                    table_hbm.at[idx_slice(c)], rows[b], sems[b]
                ).wait()

                def grp_body(g, carry):
                    ps = []
                    for i in range(_L):
                        acc = rows[b][g * _L + i, pl.ds(0, _L)]
                        for k in range(1, F // _L):
                            acc = acc + rows[b][g * _L + i, pl.ds(k * _L, _L)]
                        ps.append(acc)
                    for s in (1, 2, 4, 8):
                        ps = [merge(ps[2 * i], ps[2 * i + 1], s)
                              for i in range(len(ps) // 2)]
                    out_v[c // cpr,
                          pl.ds((c % cpr) * _CH + g * _L, _L)] = (
                        ps[0] * (1.0 / F))
                    return carry

                lax.fori_loop(0, _G, grp_body, 0)

                @pl.when(co < _NCHUNK // _NB - 1)
                def _():
                    pltpu.async_copy(
                        table_hbm.at[idx_slice(c + _NB)], rows[b], sems[b]
                    )
            return carry

        lax.fori_loop(0, _NCHUNK // _NB, outer, 0)
        pltpu.sync_copy(out_v,
                        out_hbm.at[pl.ds(wid * (_PW // _IR), _PW // _IR)])

    return _sc_pool


def _head_body(p_ref, w_ref, b_ref, g_ref, be_ref, o_ref):
    p = p_ref[:]
    y = lax.dot_general(
        p, w_ref[:], (((1,), (1,)), ((), ())),
        preferred_element_type=jnp.float32,
    )
    y = y + b_ref[:]
    mu = jnp.mean(y, axis=0, keepdims=True)
    yc = y - mu
    var = jnp.mean(yc * yc, axis=0, keepdims=True)
    y = yc * lax.rsqrt(var + EPS) * g_ref[:] + be_ref[:]
    mu_r = jnp.mean(y, axis=1, keepdims=True)
    yr = y - mu_r
    var_r = jnp.mean(yr * yr, axis=1, keepdims=True)
    o_ref[:] = yr * lax.rsqrt(var_r + EPS)


def _head(pooled, W, b2, g2, be2):
    return pl.pallas_call(
        _head_body,
        out_shape=jax.ShapeDtypeStruct((B, F), jnp.float32),
    )(pooled, W, b2, g2, be2)


def kernel(x, table, W, b, gamma, beta):
    idx = x.astype(jnp.int32)
    pooled = _make_sc_pool()(table, idx)
    return _head(
        pooled, W, b.reshape(1, F), gamma.reshape(1, F), beta.reshape(1, F)
    )


# speedup vs baseline: 1.4096x; 1.4096x over previous
import functools

import jax
import jax.numpy as jnp
from jax import lax
from jax.experimental import pallas as pl
from jax.experimental.pallas import tpu as pltpu
from jax.experimental.pallas import tpu_sc as plsc

V = 1_000_000
F = 128
B = 4096
EPS = 1e-5

_NC, _NS = 2, 16
_NW = _NC * _NS
_PW = (B * F) // _NW
_CH = 128
_NCHUNK = _PW // _CH
_NB = 4
_L = 16
_G = _CH // _L


@functools.cache
def _make_sc_pool():
    mesh = plsc.VectorSubcoreMesh(core_axis_name="c", subcore_axis_name="s")

    @functools.partial(
        pl.kernel,
        out_type=jax.ShapeDtypeStruct((B, F), jnp.float32),
        mesh=mesh,
        compiler_params=pltpu.CompilerParams(needs_layout_passes=False),
        scratch_types=[
            pltpu.VMEM((_NCHUNK, _CH), jnp.int32),
            pltpu.VMEM((_CH, F), jnp.float32),
            pltpu.VMEM((_CH, F), jnp.float32),
            pltpu.VMEM((_CH, F), jnp.float32),
            pltpu.VMEM((_CH, F), jnp.float32),
            pltpu.VMEM((_NCHUNK, _CH), jnp.float32),
            pltpu.SemaphoreType.DMA,
            pltpu.SemaphoreType.DMA,
            pltpu.SemaphoreType.DMA,
            pltpu.SemaphoreType.DMA,
        ],
    )
    def _sc_pool(table_hbm, idx_hbm, out_hbm, idx_v, r0, r1, r2, r3, out_v,
                 s0, s1, s2, s3):
        sems = (s0, s1, s2, s3)
        rows = (r0, r1, r2, r3)
        wid = lax.axis_index("s") * _NC + lax.axis_index("c")
        pltpu.sync_copy(idx_hbm.at[pl.ds(wid * _NCHUNK, _NCHUNK)], idx_v)

        lanes = lax.broadcasted_iota(jnp.int32, (_L,), 0)

        def take16(v, idx):
            return jnp.take_along_axis(v, idx, axis=0,
                                       mode="promise_in_bounds")

        def merge(a, b, s):
            sel = (lanes & s) == 0
            return jnp.where(sel, a + take16(a, lanes ^ s),
                             b + take16(b, lanes ^ s))

        for b in range(_NB):
            pltpu.async_copy(
                table_hbm.at[idx_v.at[b]], rows[b], sems[b]
            )

        def outer(co, carry):
            for b in range(_NB):
                c = co * _NB + b
                pltpu.make_async_copy(
                    table_hbm.at[idx_v.at[c]], rows[b], sems[b]
                ).wait()

                def grp_body(g, carry):
                    ps = []
                    for i in range(_L):
                        acc = rows[b][g * _L + i, pl.ds(0, _L)]
                        for k in range(1, F // _L):
                            acc = acc + rows[b][g * _L + i, pl.ds(k * _L, _L)]
                        ps.append(acc)
                    for s in (1, 2, 4, 8):
                        ps = [merge(ps[2 * i], ps[2 * i + 1], s)
                              for i in range(len(ps) // 2)]
                    out_v[c, pl.ds(g * _L, _L)] = ps[0] * (1.0 / F)
                    return carry

                lax.fori_loop(0, _G, grp_body, 0)

                @pl.when(co < _NCHUNK // _NB - 1)
                def _():
                    pltpu.async_copy(
                        table_hbm.at[idx_v.at[c + _NB]], rows[b], sems[b]
                    )
            return carry

        lax.fori_loop(0, _NCHUNK // _NB, outer, 0)
        pltpu.sync_copy(out_v, out_hbm.at[pl.ds(wid * _NCHUNK, _NCHUNK)])

    return _sc_pool


def _head_body(p_ref, w_ref, b_ref, g_ref, be_ref, o_ref):
    p = p_ref[:]
    y = lax.dot_general(
        p, w_ref[:], (((1,), (1,)), ((), ())),
        preferred_element_type=jnp.float32,
    )
    y = y + b_ref[:][None, :]
    mu = jnp.mean(y, axis=0, keepdims=True)
    yc = y - mu
    var = jnp.mean(yc * yc, axis=0, keepdims=True)
    y = yc * lax.rsqrt(var + EPS) * g_ref[:][None, :] + be_ref[:][None, :]
    mu_r = jnp.mean(y, axis=1, keepdims=True)
    yr = y - mu_r
    var_r = jnp.mean(yr * yr, axis=1, keepdims=True)
    o_ref[:] = yr * lax.rsqrt(var_r + EPS)


def _head(pooled, W, b2, g2, be2):
    return pl.pallas_call(
        _head_body,
        out_shape=jax.ShapeDtypeStruct((B, F), jnp.float32),
    )(pooled, W, b2, g2, be2)


def kernel(x, table, W, b, gamma, beta):
    idx = x.astype(jnp.int32)
    pooled = _make_sc_pool()(table, idx)
    return _head(pooled, W, b, gamma, beta)
